# baseline (speedup 1.0000x reference)
import functools
import math

import numpy as np

import jax
import jax.numpy as jnp
from jax import lax
from jax.experimental import pallas as pl
from jax.experimental.pallas import tpu as pltpu
from jax.experimental.pallas import tpu_sc as plsc

N = 50000
E = 800000
NMOL = 500
HIDDEN = 64
NRBF = 50
CUTOFF = 5.0
NINTER = 3
MAXZ = 100

BN = 1000
BE = 4096
E_PAD = 802816
NBN = N // BN
NBE = E_PAD // BE

_LN2 = math.log(2.0)

_PERM32 = np.stack([np.arange(16), np.arange(16) + 16], axis=1).reshape(-1)
F_PERM = np.concatenate([_PERM32, _PERM32 + 32])


def _ssp(x):
    return jnp.maximum(x, 0.0) + jnp.log(1.0 + jnp.exp(-jnp.abs(x))) - _LN2


def _dot(a, b):
    return jax.lax.dot_general(a, b, (((1,), (0,)), ((), ())),
                               preferred_element_type=jnp.float32)


def _embed_body(z_ref, emb_ref, w_ref, x0_ref, f0_ref):
    zb = z_ref[0, 0, :]
    oh = (zb[:, None] == lax.broadcasted_iota(jnp.int32, (BN, MAXZ), 1))
    oh = oh.astype(jnp.float32)
    x0 = _dot(oh, emb_ref[...])
    x0_ref[...] = x0
    f0_ref[...] = _dot(x0, w_ref[...]).astype(jnp.bfloat16)


def _embed(z3, emb, w0):
    return pl.pallas_call(
        _embed_body,
        grid=(NBN,),
        in_specs=[
            pl.BlockSpec((1, 1, BN), lambda i: (i, 0, 0)),
            pl.BlockSpec((MAXZ, HIDDEN), lambda i: (0, 0)),
            pl.BlockSpec((HIDDEN, HIDDEN), lambda i: (0, 0)),
        ],
        out_specs=[
            pl.BlockSpec((BN, HIDDEN), lambda i: (i, 0)),
            pl.BlockSpec((BN, HIDDEN), lambda i: (i, 0)),
        ],
        out_shape=[
            jax.ShapeDtypeStruct((N, HIDDEN), jnp.float32),
            jax.ShapeDtypeStruct((N, HIDDEN), jnp.bfloat16),
        ],
    )(z3, emb, w0)


def _dist_body(pi_ref, pj_ref, d_ref):
    diff = (pj_ref[...] - pi_ref[...]).reshape(BE, 8)
    d2 = jnp.sum(diff * diff, axis=1) + 1e-8
    d_ref[0, 0, :] = jnp.sqrt(d2)


def _dist(pi3, pj3):
    nb = BE // PCH
    return pl.pallas_call(
        _dist_body,
        grid=(NBE,),
        in_specs=[
            pl.BlockSpec((nb, PCH, 8), lambda i: (i, 0, 0)),
            pl.BlockSpec((nb, PCH, 8), lambda i: (i, 0, 0)),
        ],
        out_specs=pl.BlockSpec((1, 1, BE), lambda i: (i, 0, 0)),
        out_shape=jax.ShapeDtypeStruct((NBE, 1, BE), jnp.float32),
    )(pi3, pj3)


def _filter_body(d_ref, w1_ref, b1_ref, w2_ref, b2_ref, wf_ref):
    d = jnp.sqrt(d_ref[0, 0, :] + 1e-8)
    step = CUTOFF / (NRBF - 1)
    coeff = -0.5 / (step * step)
    offs = lax.broadcasted_iota(jnp.int32, (BE, NRBF), 1).astype(jnp.float32) * step
    delta = d[:, None] - offs
    rbf = jnp.exp(coeff * delta * delta)
    h = _ssp(_dot(rbf, w1_ref[...]) + b1_ref[...])
    wf = _dot(h, w2_ref[...]) + b2_ref[...]
    fcut = 0.5 * (jnp.cos(jnp.pi * d / CUTOFF) + 1.0)
    fcut = fcut * (d < CUTOFF).astype(jnp.float32)
    wf_ref[...] = (wf * fcut[:, None]).reshape(BE // CHUNK, CHUNK, HIDDEN)


def _filter(d3, w1, b1, w2, b2):
    return pl.pallas_call(
        _filter_body,
        grid=(NBE,),
        in_specs=[
            pl.BlockSpec((1, 1, BE), lambda i: (i, 0, 0)),
            pl.BlockSpec((NRBF, HIDDEN), lambda i: (0, 0)),
            pl.BlockSpec((1, HIDDEN), lambda i: (0, 0)),
            pl.BlockSpec((HIDDEN, HIDDEN), lambda i: (0, 0)),
            pl.BlockSpec((1, HIDDEN), lambda i: (0, 0)),
        ],
        out_specs=pl.BlockSpec((BE // CHUNK, CHUNK, HIDDEN),
                               lambda i: (i, 0, 0)),
        out_shape=jax.ShapeDtypeStruct((E_PAD // CHUNK, CHUNK, HIDDEN),
                                       jnp.float32),
    )(d3, w1, b1, w2, b2)


def _node_body(has_next, agg_ref, x_ref, w1_ref, b1_ref, w2_ref, b2_ref,
               wn_ref, xn_ref, *maybe_f):
    h = _ssp(_dot(agg_ref[...], w1_ref[...]) + b1_ref[...])
    v = _dot(h, w2_ref[...]) + b2_ref[...]
    xn = x_ref[...] + v
    xn_ref[...] = xn
    if has_next:
        maybe_f[0][...] = _dot(xn, wn_ref[...]).astype(jnp.bfloat16)


def _node(agg, x, w1, b1, w2, b2, wnext, has_next):
    out_specs = [pl.BlockSpec((BN, HIDDEN), lambda i: (i, 0))]
    out_shape = [jax.ShapeDtypeStruct((N, HIDDEN), jnp.float32)]
    if has_next:
        out_specs.append(pl.BlockSpec((BN, HIDDEN), lambda i: (i, 0)))
        out_shape.append(jax.ShapeDtypeStruct((N, HIDDEN), jnp.bfloat16))
    res = pl.pallas_call(
        functools.partial(_node_body, has_next),
        grid=(NBN,),
        in_specs=[
            pl.BlockSpec((BN, HIDDEN), lambda i: (i, 0)),
            pl.BlockSpec((BN, HIDDEN), lambda i: (i, 0)),
            pl.BlockSpec((HIDDEN, HIDDEN), lambda i: (0, 0)),
            pl.BlockSpec((1, HIDDEN), lambda i: (0, 0)),
            pl.BlockSpec((HIDDEN, HIDDEN), lambda i: (0, 0)),
            pl.BlockSpec((1, HIDDEN), lambda i: (0, 0)),
            pl.BlockSpec((HIDDEN, HIDDEN), lambda i: (0, 0)),
        ],
        out_specs=out_specs,
        out_shape=out_shape,
    )(agg, x, w1, b1, w2, b2, wnext)
    return res if has_next else (res[0], None)


def _readout_body(x_ref, b_ref, w1_ref, b1_ref, w2r_ref, b2_ref, e_ref):
    i = pl.program_id(0)
    h = _ssp(_dot(x_ref[...], w1_ref[...]) + b1_ref[...])
    yi = jnp.sum(h * w2r_ref[...], axis=1) + b2_ref[0, 0]
    bb = b_ref[0, 0, :]
    oh = (bb[:, None] == lax.broadcasted_iota(jnp.int32, (BN, 512), 1))
    contrib = _dot(yi[None, :], oh.astype(jnp.float32))

    @pl.when(i == 0)
    def _init():
        e_ref[...] = jnp.zeros_like(e_ref)

    e_ref[...] += contrib


def _readout(x, b3, aw1, aw1b, aw2r, aw2b):
    return pl.pallas_call(
        _readout_body,
        grid=(NBN,),
        in_specs=[
            pl.BlockSpec((BN, HIDDEN), lambda i: (i, 0)),
            pl.BlockSpec((1, 1, BN), lambda i: (i, 0, 0)),
            pl.BlockSpec((HIDDEN, 32), lambda i: (0, 0)),
            pl.BlockSpec((1, 32), lambda i: (0, 0)),
            pl.BlockSpec((1, 32), lambda i: (0, 0)),
            pl.BlockSpec((1, 1), lambda i: (0, 0)),
        ],
        out_specs=pl.BlockSpec((1, 512), lambda i: (0, 0)),
        out_shape=jax.ShapeDtypeStruct((1, 512), jnp.float32),
    )(x, b3, aw1, aw1b, aw2r, aw2b)


HALF = 25088
AGG_PAD = 2 * HALF
ACC_ROWS = HALF + 8
EPT = E_PAD // 16
CHUNK = 64
NCHUNK = EPT // CHUNK
GRP = 16
NGRP = NCHUNK // GRP
RPT = HALF // 16
ZROWS = 32


def _sc_gather_mul_scatter(f, wf, idxi2, idxj2):
    mesh = plsc.VectorSubcoreMesh(core_axis_name="c", subcore_axis_name="s")

    @functools.partial(
        pl.kernel, mesh=mesh,
        compiler_params=pltpu.CompilerParams(use_tc_tiling_on_sc=False,
                                             needs_layout_passes=False),
        out_type=jax.ShapeDtypeStruct((AGG_PAD, HIDDEN), jnp.float32),
        scratch_types=[
            pltpu.VMEM((GRP, CHUNK), jnp.int32),
            pltpu.VMEM((GRP, CHUNK), jnp.int32),
            pltpu.VMEM((2, CHUNK, HIDDEN), jnp.float32),
            pltpu.VMEM((2, CHUNK, HIDDEN), jnp.bfloat16),
            pltpu.VMEM((CHUNK, HIDDEN), jnp.float32),
            pltpu.VMEM((CHUNK,), jnp.int32),
            pltpu.VMEM((ZROWS, HIDDEN), jnp.float32),
            pltpu.VMEM_SHARED((ACC_ROWS, HIDDEN), jnp.float32),
            pltpu.SemaphoreType.DMA,
            pltpu.SemaphoreType.DMA,
            pltpu.SemaphoreType.DMA,
            pltpu.SemaphoreType.DMA,
        ],
    )
    def k(f_hbm, wf_hbm, ii_hbm, jj_hbm, out_hbm,
          idxi_b, idxj_b, wf_b, rows_b, msg_b, sidx, zbuf, acc,
          gs0, gs1, ws0, ws1):
        cid = lax.axis_index("c")
        sid = lax.axis_index("s")
        base = cid * HALF
        row0 = sid * RPT

        def zb(i, c):
            zbuf[i // 4, pl.ds((i % 4) * 16, 16)] = jnp.zeros((16,), jnp.float32)
            return c
        lax.fori_loop(0, ZROWS * 4, zb, 0)

        def zc(kk, c):
            pltpu.sync_copy(zbuf, acc.at[pl.ds(row0 + kk * ZROWS, ZROWS), :])
            return c
        lax.fori_loop(0, RPT // ZROWS, zc, 0)
        plsc.subcore_barrier()

        gsems = (gs0, gs1)
        wsems = (ws0, ws1)
        irow0 = sid * NCHUNK

        def group(g, c):
            pltpu.sync_copy(ii_hbm.at[pl.ds(irow0 + g * GRP, GRP), :], idxi_b)
            pltpu.sync_copy(jj_hbm.at[pl.ds(irow0 + g * GRP, GRP), :], idxj_b)
            c0 = irow0 + g * GRP
            handles = {}

            def issue(b):
                slot = b % 2
                hg = pltpu.async_copy(f_hbm.at[idxj_b.at[b]],
                                      rows_b.at[slot], gsems[slot])
                hw = pltpu.async_copy(wf_hbm.at[c0 + b],
                                      wf_b.at[slot], wsems[slot])
                handles[b] = (hg, hw)

            issue(0)
            for b in range(GRP):
                if b + 1 < GRP:
                    issue(b + 1)
                hg, hw = handles[b]
                hg.wait()
                hw.wait()
                slot = b % 2
                for v in range(CHUNK // 16):
                    iv = idxi_b[b, pl.ds(v * 16, 16)]
                    loc = iv - base
                    ok = (loc >= 0) & (loc < HALF)
                    sidx[pl.ds(v * 16, 16)] = jnp.where(ok, loc, HALF)

                def mul(r, c2):
                    for half in range(2):
                        rb = rows_b[slot, r, pl.ds(half * 32, 32)]
                        a, b2 = plsc.unpack(rb,
                                            format=plsc.PackFormat.INTERLEAVED)
                        sl0 = pl.ds(half * 32, 16)
                        sl1 = pl.ds(half * 32 + 16, 16)
                        msg_b[r, sl0] = a.astype(jnp.float32) * wf_b[slot, r, sl0]
                        msg_b[r, sl1] = b2.astype(jnp.float32) * wf_b[slot, r, sl1]
                    return c2
                lax.fori_loop(0, CHUNK, mul, 0)
                pltpu.sync_copy(msg_b, acc.at[sidx], add=True)
            return c
        lax.fori_loop(0, NGRP, group, 0)
        plsc.subcore_barrier()
        pltpu.sync_copy(acc.at[pl.ds(row0, RPT), :],
                        out_hbm.at[pl.ds(base + row0, RPT), :])

    return k(f, wf, idxi2, idxj2)


PCH = 128
PNCH = E_PAD // (32 * PCH)
PGRP = 14
PNGRP = PNCH // PGRP


def _take16(x, idx):
    dn = lax.GatherDimensionNumbers(offset_dims=(), collapsed_slice_dims=(0,),
                                    start_index_map=(0,))
    return lax.gather(x, idx[:, None], dn, (1,),
                      mode=lax.GatherScatterMode.PROMISE_IN_BOUNDS)


def _sc_dist2(pos8, idxi2p, idxj2p):
    mesh = plsc.VectorSubcoreMesh(core_axis_name="c", subcore_axis_name="s")

    @functools.partial(
        pl.kernel, mesh=mesh,
        compiler_params=pltpu.CompilerParams(use_tc_tiling_on_sc=False,
                                             needs_layout_passes=False),
        out_type=jax.ShapeDtypeStruct((E_PAD // PCH, PCH), jnp.float32),
        scratch_types=[
            pltpu.VMEM((PGRP, PCH), jnp.int32),
            pltpu.VMEM((PGRP, PCH), jnp.int32),
            pltpu.VMEM((2, PCH, 16), jnp.float32),
            pltpu.VMEM((2, PCH, 16), jnp.float32),
            pltpu.VMEM((PCH,), jnp.float32),
            pltpu.SemaphoreType.DMA,
            pltpu.SemaphoreType.DMA,
            pltpu.SemaphoreType.DMA,
            pltpu.SemaphoreType.DMA,
        ],
    )
    def k(pos_hbm, ii_hbm, jj_hbm, od_hbm,
          idxi_b, idxj_b, pi_b, pj_b, d2_v, s0, s1, s2, s3):
        cid = lax.axis_index("c")
        sid = lax.axis_index("s")
        wid = cid * 16 + sid
        irow0 = wid * PNCH
        isems = (s0, s1)
        jsems = (s2, s3)
        lane = lax.iota(jnp.int32, 16)
        p2 = lane ^ 2
        p1 = lane ^ 1
        pack_mask = lane == 0

        def group(g, c):
            pltpu.sync_copy(ii_hbm.at[pl.ds(irow0 + g * PGRP, PGRP), :], idxi_b)
            pltpu.sync_copy(jj_hbm.at[pl.ds(irow0 + g * PGRP, PGRP), :], idxj_b)
            c0 = irow0 + g * PGRP
            handles = {}

            def issue(b):
                slot = b % 2
                hi = pltpu.async_copy(pos_hbm.at[idxi_b.at[b]],
                                      pi_b.at[slot], isems[slot])
                hj = pltpu.async_copy(pos_hbm.at[idxj_b.at[b]],
                                      pj_b.at[slot], jsems[slot])
                handles[b] = (hi, hj)

            issue(0)
            for b in range(PGRP):
                if b + 1 < PGRP:
                    issue(b + 1)
                hi, hj = handles[b]
                hi.wait()
                hj.wait()
                slot = b % 2

                def edge2(v, c2):
                    x = pj_b[slot, v, :] - pi_b[slot, v, :]
                    s = x * x
                    s = s + _take16(s, p1)
                    s = s + _take16(s, p2)
                    tgt = lane * 0 + v
                    plsc.store_scatter(d2_v, [tgt], s, mask=pack_mask)
                    return c2
                lax.fori_loop(0, PCH, edge2, 0)
                pltpu.sync_copy(d2_v, od_hbm.at[c0 + b])
            return c
        lax.fori_loop(0, PNGRP, group, 0)

    return k(pos8, idxi2p, idxj2p)


def kernel(z, pos, batch, edge_index, emb, in2f_W, fn1_W, fn1_b, fn2_W, fn2_b,
           f2out1_W, f2out1_b, f2out2_W, f2out2_b, aw1_W, aw1_b, aw2_W, aw2_b):
    z = z.astype(jnp.int32)
    batch = batch.astype(jnp.int32)
    edge_index = edge_index.astype(jnp.int32)
    idx_i = edge_index[0]
    idx_j = edge_index[1]
    pad = E_PAD - E
    idx_i_p = jnp.pad(idx_i, (0, pad), constant_values=1 << 20)
    idx_j_p = jnp.pad(idx_j, (0, pad), constant_values=0)

    pos16 = jnp.pad(pos, ((0, 0), (0, 13)))

    idx_i_g = jnp.pad(idx_i, (0, pad), constant_values=0)
    d2 = _sc_dist2(pos16,
                   idx_i_g.reshape(E_PAD // PCH, PCH),
                   idx_j_p.reshape(E_PAD // PCH, PCH))
    d3 = d2.reshape(NBE, 1, BE)

    z3 = z.reshape(NBN, 1, BN)
    b3 = batch.reshape(NBN, 1, BN)

    w_in = in2f_W[:, :, jnp.asarray(F_PERM)]
    x, f = _embed(z3, emb, w_in[0])

    idxi2 = idx_i_p.reshape(E_PAD // CHUNK, CHUNK)
    idxj2 = idx_j_p.reshape(E_PAD // CHUNK, CHUNK)

    for t in range(NINTER):
        wf = _filter(d3, fn1_W[t], fn1_b[t].reshape(1, HIDDEN),
                     fn2_W[t], fn2_b[t].reshape(1, HIDDEN))
        agg = _sc_gather_mul_scatter(f, wf, idxi2, idxj2)[:N]
        has_next = t + 1 < NINTER
        wnext = w_in[t + 1] if has_next else w_in[0]
        x, f = _node(agg, x, f2out1_W[t], f2out1_b[t].reshape(1, HIDDEN),
                     f2out2_W[t], f2out2_b[t].reshape(1, HIDDEN),
                     wnext, has_next)

    e = _readout(x, b3, aw1_W, aw1_b.reshape(1, 32),
                 aw2_W.reshape(1, 32), aw2_b.reshape(1, 1))
    return e[0, :NMOL]


# speedup vs baseline: 1.1521x; 1.1521x over previous
import functools
import math

import numpy as np

import jax
import jax.numpy as jnp
from jax import lax
from jax.experimental import pallas as pl
from jax.experimental.pallas import tpu as pltpu
from jax.experimental.pallas import tpu_sc as plsc

N = 50000
E = 800000
NMOL = 500
HIDDEN = 64
NRBF = 50
CUTOFF = 5.0
NINTER = 3
MAXZ = 100

BN = 1000
BE = 4096
E_PAD = 802816
NBN = N // BN
NBE = E_PAD // BE

_LN2 = math.log(2.0)

_PERM32 = np.stack([np.arange(16), np.arange(16) + 16], axis=1).reshape(-1)
F_PERM = np.concatenate([_PERM32, _PERM32 + 32])


def _ssp(x):
    return jnp.maximum(x, 0.0) + jnp.log(1.0 + jnp.exp(-jnp.abs(x))) - _LN2


def _dot(a, b):
    return jax.lax.dot_general(a, b, (((1,), (0,)), ((), ())),
                               preferred_element_type=jnp.float32)


def _embed_body(z_ref, emb_ref, w_ref, x0_ref, f0_ref):
    zb = z_ref[0, 0, :]
    oh = (zb[:, None] == lax.broadcasted_iota(jnp.int32, (BN, MAXZ), 1))
    oh = oh.astype(jnp.float32)
    x0 = _dot(oh, emb_ref[...])
    x0_ref[...] = x0
    f0_ref[...] = _dot(x0, w_ref[...])


def _embed(z3, emb, w0):
    return pl.pallas_call(
        _embed_body,
        grid=(NBN,),
        in_specs=[
            pl.BlockSpec((1, 1, BN), lambda i: (i, 0, 0)),
            pl.BlockSpec((MAXZ, HIDDEN), lambda i: (0, 0)),
            pl.BlockSpec((HIDDEN, HIDDEN), lambda i: (0, 0)),
        ],
        out_specs=[
            pl.BlockSpec((BN, HIDDEN), lambda i: (i, 0)),
            pl.BlockSpec((BN, HIDDEN), lambda i: (i, 0)),
        ],
        out_shape=[
            jax.ShapeDtypeStruct((N, HIDDEN), jnp.float32),
            jax.ShapeDtypeStruct((N, HIDDEN), jnp.float32),
        ],
    )(z3, emb, w0)


def _dist_body(pi_ref, pj_ref, d_ref):
    diff = (pj_ref[...] - pi_ref[...]).reshape(BE, 8)
    d2 = jnp.sum(diff * diff, axis=1) + 1e-8
    d_ref[0, 0, :] = jnp.sqrt(d2)


def _dist(pi3, pj3):
    nb = BE // PCH
    return pl.pallas_call(
        _dist_body,
        grid=(NBE,),
        in_specs=[
            pl.BlockSpec((nb, PCH, 8), lambda i: (i, 0, 0)),
            pl.BlockSpec((nb, PCH, 8), lambda i: (i, 0, 0)),
        ],
        out_specs=pl.BlockSpec((1, 1, BE), lambda i: (i, 0, 0)),
        out_shape=jax.ShapeDtypeStruct((NBE, 1, BE), jnp.float32),
    )(pi3, pj3)


def _filter_body(d_ref, w1_ref, b1_ref, w2_ref, b2_ref, wf_ref):
    d = jnp.sqrt(d_ref[0, 0, :] + 1e-8)
    step = CUTOFF / (NRBF - 1)
    coeff = -0.5 / (step * step)
    offs = lax.broadcasted_iota(jnp.int32, (BE, NRBF), 1).astype(jnp.float32) * step
    delta = d[:, None] - offs
    rbf = jnp.exp(coeff * delta * delta)
    h = _ssp(_dot(rbf, w1_ref[...]) + b1_ref[...])
    wf = _dot(h, w2_ref[...]) + b2_ref[...]
    fcut = 0.5 * (jnp.cos(jnp.pi * d / CUTOFF) + 1.0)
    fcut = fcut * (d < CUTOFF).astype(jnp.float32)
    wf_ref[...] = (wf * fcut[:, None]).reshape(BE // CHUNK, CHUNK, HIDDEN)


def _filter(d3, w1, b1, w2, b2):
    return pl.pallas_call(
        _filter_body,
        grid=(NBE,),
        in_specs=[
            pl.BlockSpec((1, 1, BE), lambda i: (i, 0, 0)),
            pl.BlockSpec((NRBF, HIDDEN), lambda i: (0, 0)),
            pl.BlockSpec((1, HIDDEN), lambda i: (0, 0)),
            pl.BlockSpec((HIDDEN, HIDDEN), lambda i: (0, 0)),
            pl.BlockSpec((1, HIDDEN), lambda i: (0, 0)),
        ],
        out_specs=pl.BlockSpec((BE // CHUNK, CHUNK, HIDDEN),
                               lambda i: (i, 0, 0)),
        out_shape=jax.ShapeDtypeStruct((E_PAD // CHUNK, CHUNK, HIDDEN),
                                       jnp.float32),
    )(d3, w1, b1, w2, b2)


def _node_body(has_next, agg_ref, x_ref, w1_ref, b1_ref, w2_ref, b2_ref,
               wn_ref, xn_ref, *maybe_f):
    h = _ssp(_dot(agg_ref[...], w1_ref[...]) + b1_ref[...])
    v = _dot(h, w2_ref[...]) + b2_ref[...]
    xn = x_ref[...] + v
    xn_ref[...] = xn
    if has_next:
        maybe_f[0][...] = _dot(xn, wn_ref[...])


def _node(agg, x, w1, b1, w2, b2, wnext, has_next):
    out_specs = [pl.BlockSpec((BN, HIDDEN), lambda i: (i, 0))]
    out_shape = [jax.ShapeDtypeStruct((N, HIDDEN), jnp.float32)]
    if has_next:
        out_specs.append(pl.BlockSpec((BN, HIDDEN), lambda i: (i, 0)))
        out_shape.append(jax.ShapeDtypeStruct((N, HIDDEN), jnp.float32))
    res = pl.pallas_call(
        functools.partial(_node_body, has_next),
        grid=(NBN,),
        in_specs=[
            pl.BlockSpec((BN, HIDDEN), lambda i: (i, 0)),
            pl.BlockSpec((BN, HIDDEN), lambda i: (i, 0)),
            pl.BlockSpec((HIDDEN, HIDDEN), lambda i: (0, 0)),
            pl.BlockSpec((1, HIDDEN), lambda i: (0, 0)),
            pl.BlockSpec((HIDDEN, HIDDEN), lambda i: (0, 0)),
            pl.BlockSpec((1, HIDDEN), lambda i: (0, 0)),
            pl.BlockSpec((HIDDEN, HIDDEN), lambda i: (0, 0)),
        ],
        out_specs=out_specs,
        out_shape=out_shape,
    )(agg, x, w1, b1, w2, b2, wnext)
    return res if has_next else (res[0], None)


def _readout_body(x_ref, b_ref, w1_ref, b1_ref, w2r_ref, b2_ref, e_ref):
    i = pl.program_id(0)
    h = _ssp(_dot(x_ref[...], w1_ref[...]) + b1_ref[...])
    yi = jnp.sum(h * w2r_ref[...], axis=1) + b2_ref[0, 0]
    bb = b_ref[0, 0, :]
    oh = (bb[:, None] == lax.broadcasted_iota(jnp.int32, (BN, 512), 1))
    contrib = _dot(yi[None, :], oh.astype(jnp.float32))

    @pl.when(i == 0)
    def _init():
        e_ref[...] = jnp.zeros_like(e_ref)

    e_ref[...] += contrib


def _readout(x, b3, aw1, aw1b, aw2r, aw2b):
    return pl.pallas_call(
        _readout_body,
        grid=(NBN,),
        in_specs=[
            pl.BlockSpec((BN, HIDDEN), lambda i: (i, 0)),
            pl.BlockSpec((1, 1, BN), lambda i: (i, 0, 0)),
            pl.BlockSpec((HIDDEN, 32), lambda i: (0, 0)),
            pl.BlockSpec((1, 32), lambda i: (0, 0)),
            pl.BlockSpec((1, 32), lambda i: (0, 0)),
            pl.BlockSpec((1, 1), lambda i: (0, 0)),
        ],
        out_specs=pl.BlockSpec((1, 512), lambda i: (0, 0)),
        out_shape=jax.ShapeDtypeStruct((1, 512), jnp.float32),
    )(x, b3, aw1, aw1b, aw2r, aw2b)


HALF = 25088
AGG_PAD = 2 * HALF
ACC_ROWS = HALF + 8
EPT = E_PAD // 16
CHUNK = 64
NCHUNK = EPT // CHUNK
GRP = 16
NGRP = NCHUNK // GRP
RPT = HALF // 16
ZROWS = 32


def _sc_gather_mul_scatter(f, wf, idxi2, idxj2):
    mesh = plsc.VectorSubcoreMesh(core_axis_name="c", subcore_axis_name="s")

    @functools.partial(
        pl.kernel, mesh=mesh,
        compiler_params=pltpu.CompilerParams(use_tc_tiling_on_sc=False,
                                             needs_layout_passes=False),
        out_type=jax.ShapeDtypeStruct((AGG_PAD, HIDDEN), jnp.float32),
        scratch_types=[
            pltpu.VMEM((GRP, CHUNK), jnp.int32),
            pltpu.VMEM((GRP, CHUNK), jnp.int32),
            pltpu.VMEM((3, CHUNK, HIDDEN), jnp.float32),
            pltpu.VMEM((3, CHUNK, HIDDEN), jnp.float32),
            pltpu.VMEM((CHUNK,), jnp.int32),
            pltpu.VMEM((ZROWS, HIDDEN), jnp.float32),
            pltpu.VMEM_SHARED((ACC_ROWS, HIDDEN), jnp.float32),
            pltpu.SemaphoreType.DMA,
            pltpu.SemaphoreType.DMA,
            pltpu.SemaphoreType.DMA,
            pltpu.SemaphoreType.DMA,
            pltpu.SemaphoreType.DMA,
            pltpu.SemaphoreType.DMA,
        ],
    )
    def k(f_hbm, wf_hbm, ii_hbm, jj_hbm, out_hbm,
          idxi_b, idxj_b, wf_b, rows_b, sidx, zbuf, acc,
          gs0, gs1, gs2, ws0, ws1, ws2):
        cid = lax.axis_index("c")
        sid = lax.axis_index("s")
        base = cid * HALF
        row0 = sid * RPT

        def zb(i, c):
            zbuf[i // 4, pl.ds((i % 4) * 16, 16)] = jnp.zeros((16,), jnp.float32)
            return c
        lax.fori_loop(0, ZROWS * 4, zb, 0)

        def zc(kk, c):
            pltpu.sync_copy(zbuf, acc.at[pl.ds(row0 + kk * ZROWS, ZROWS), :])
            return c
        lax.fori_loop(0, RPT // ZROWS, zc, 0)
        plsc.subcore_barrier()

        gsems = (gs0, gs1, gs2)
        wsems = (ws0, ws1, ws2)
        irow0 = sid * NCHUNK

        def group(g, c):
            pltpu.sync_copy(ii_hbm.at[pl.ds(irow0 + g * GRP, GRP), :], idxi_b)
            pltpu.sync_copy(jj_hbm.at[pl.ds(irow0 + g * GRP, GRP), :], idxj_b)
            c0 = irow0 + g * GRP
            handles = {}

            def issue(b):
                slot = b % 3
                hg = pltpu.async_copy(f_hbm.at[idxj_b.at[b]],
                                      rows_b.at[slot], gsems[slot])
                hw = pltpu.async_copy(wf_hbm.at[c0 + b],
                                      wf_b.at[slot], wsems[slot])
                handles[b] = (hg, hw)

            issue(0)
            issue(1)
            for b in range(GRP):
                if b + 2 < GRP:
                    issue(b + 2)
                hg, hw = handles[b]
                hg.wait()
                hw.wait()
                slot = b % 3
                for v in range(CHUNK // 16):
                    iv = idxi_b[b, pl.ds(v * 16, 16)]
                    loc = iv - base
                    ok = (loc >= 0) & (loc < HALF)
                    sidx[pl.ds(v * 16, 16)] = jnp.where(ok, loc, HALF)

                def mul(r, c2):
                    for sgm in range(4):
                        sl = pl.ds(sgm * 16, 16)
                        rows_b[slot, r, sl] = (rows_b[slot, r, sl]
                                               * wf_b[slot, r, sl])
                    return c2
                lax.fori_loop(0, CHUNK, mul, 0)
                pltpu.sync_copy(rows_b.at[slot], acc.at[sidx], add=True)
            return c
        lax.fori_loop(0, NGRP, group, 0)
        plsc.subcore_barrier()
        pltpu.sync_copy(acc.at[pl.ds(row0, RPT), :],
                        out_hbm.at[pl.ds(base + row0, RPT), :])

    return k(f, wf, idxi2, idxj2)


PCH = 128
PNCH = E_PAD // (32 * PCH)
PGRP = 14
PNGRP = PNCH // PGRP


def _take16(x, idx):
    dn = lax.GatherDimensionNumbers(offset_dims=(), collapsed_slice_dims=(0,),
                                    start_index_map=(0,))
    return lax.gather(x, idx[:, None], dn, (1,),
                      mode=lax.GatherScatterMode.PROMISE_IN_BOUNDS)


def _sc_dist2(pos8, idxi2p, idxj2p):
    mesh = plsc.VectorSubcoreMesh(core_axis_name="c", subcore_axis_name="s")

    @functools.partial(
        pl.kernel, mesh=mesh,
        compiler_params=pltpu.CompilerParams(use_tc_tiling_on_sc=False,
                                             needs_layout_passes=False),
        out_type=jax.ShapeDtypeStruct((E_PAD // PCH, PCH), jnp.float32),
        scratch_types=[
            pltpu.VMEM((PGRP, PCH), jnp.int32),
            pltpu.VMEM((PGRP, PCH), jnp.int32),
            pltpu.VMEM((2, PCH, 16), jnp.float32),
            pltpu.VMEM((2, PCH, 16), jnp.float32),
            pltpu.VMEM((PCH,), jnp.float32),
            pltpu.SemaphoreType.DMA,
            pltpu.SemaphoreType.DMA,
            pltpu.SemaphoreType.DMA,
            pltpu.SemaphoreType.DMA,
        ],
    )
    def k(pos_hbm, ii_hbm, jj_hbm, od_hbm,
          idxi_b, idxj_b, pi_b, pj_b, d2_v, s0, s1, s2, s3):
        cid = lax.axis_index("c")
        sid = lax.axis_index("s")
        wid = cid * 16 + sid
        irow0 = wid * PNCH
        isems = (s0, s1)
        jsems = (s2, s3)
        lane = lax.iota(jnp.int32, 16)
        p2 = lane ^ 2
        p1 = lane ^ 1
        pack_mask = lane == 0

        def group(g, c):
            pltpu.sync_copy(ii_hbm.at[pl.ds(irow0 + g * PGRP, PGRP), :], idxi_b)
            pltpu.sync_copy(jj_hbm.at[pl.ds(irow0 + g * PGRP, PGRP), :], idxj_b)
            c0 = irow0 + g * PGRP
            handles = {}

            def issue(b):
                slot = b % 2
                hi = pltpu.async_copy(pos_hbm.at[idxi_b.at[b]],
                                      pi_b.at[slot], isems[slot])
                hj = pltpu.async_copy(pos_hbm.at[idxj_b.at[b]],
                                      pj_b.at[slot], jsems[slot])
                handles[b] = (hi, hj)

            issue(0)
            for b in range(PGRP):
                if b + 1 < PGRP:
                    issue(b + 1)
                hi, hj = handles[b]
                hi.wait()
                hj.wait()
                slot = b % 2

                def edge2(v, c2):
                    x = pj_b[slot, v, :] - pi_b[slot, v, :]
                    s = x * x
                    s = s + _take16(s, p1)
                    s = s + _take16(s, p2)
                    tgt = lane * 0 + v
                    plsc.store_scatter(d2_v, [tgt], s, mask=pack_mask)
                    return c2
                lax.fori_loop(0, PCH, edge2, 0)
                pltpu.sync_copy(d2_v, od_hbm.at[c0 + b])
            return c
        lax.fori_loop(0, PNGRP, group, 0)

    return k(pos8, idxi2p, idxj2p)


def kernel(z, pos, batch, edge_index, emb, in2f_W, fn1_W, fn1_b, fn2_W, fn2_b,
           f2out1_W, f2out1_b, f2out2_W, f2out2_b, aw1_W, aw1_b, aw2_W, aw2_b):
    z = z.astype(jnp.int32)
    batch = batch.astype(jnp.int32)
    edge_index = edge_index.astype(jnp.int32)
    idx_i = edge_index[0]
    idx_j = edge_index[1]
    pad = E_PAD - E
    idx_i_p = jnp.pad(idx_i, (0, pad), constant_values=1 << 20)
    idx_j_p = jnp.pad(idx_j, (0, pad), constant_values=0)

    pos16 = jnp.pad(pos, ((0, 0), (0, 13)))

    idx_i_g = jnp.pad(idx_i, (0, pad), constant_values=0)
    d2 = _sc_dist2(pos16,
                   idx_i_g.reshape(E_PAD // PCH, PCH),
                   idx_j_p.reshape(E_PAD // PCH, PCH))
    d3 = d2.reshape(NBE, 1, BE)

    z3 = z.reshape(NBN, 1, BN)
    b3 = batch.reshape(NBN, 1, BN)

    x, f = _embed(z3, emb, in2f_W[0])

    idxi2 = idx_i_p.reshape(E_PAD // CHUNK, CHUNK)
    idxj2 = idx_j_p.reshape(E_PAD // CHUNK, CHUNK)

    for t in range(NINTER):
        wf = _filter(d3, fn1_W[t], fn1_b[t].reshape(1, HIDDEN),
                     fn2_W[t], fn2_b[t].reshape(1, HIDDEN))
        agg = _sc_gather_mul_scatter(f, wf, idxi2, idxj2)[:N]
        has_next = t + 1 < NINTER
        wnext = in2f_W[t + 1] if has_next else in2f_W[0]
        x, f = _node(agg, x, f2out1_W[t], f2out1_b[t].reshape(1, HIDDEN),
                     f2out2_W[t], f2out2_b[t].reshape(1, HIDDEN),
                     wnext, has_next)

    e = _readout(x, b3, aw1_W, aw1_b.reshape(1, 32),
                 aw2_W.reshape(1, 32), aw2_b.reshape(1, 1))
    return e[0, :NMOL]


# speedup vs baseline: 1.1599x; 1.0068x over previous
import functools
import math

import numpy as np

import jax
import jax.numpy as jnp
from jax import lax
from jax.experimental import pallas as pl
from jax.experimental.pallas import tpu as pltpu
from jax.experimental.pallas import tpu_sc as plsc

N = 50000
E = 800000
NMOL = 500
HIDDEN = 64
NRBF = 50
CUTOFF = 5.0
NINTER = 3
MAXZ = 100

BN = 1000
BE = 4096
E_PAD = 802816
NBN = N // BN
NBE = E_PAD // BE

_LN2 = math.log(2.0)

_PERM32 = np.stack([np.arange(16), np.arange(16) + 16], axis=1).reshape(-1)
F_PERM = np.concatenate([_PERM32, _PERM32 + 32])


def _ssp(x):
    return jnp.maximum(x, 0.0) + jnp.log(1.0 + jnp.exp(-jnp.abs(x))) - _LN2


def _dot(a, b):
    return jax.lax.dot_general(a, b, (((1,), (0,)), ((), ())),
                               preferred_element_type=jnp.float32)


def _embed_body(z_ref, emb_ref, w_ref, x0_ref, f0_ref):
    zb = z_ref[0, 0, :]
    oh = (zb[:, None] == lax.broadcasted_iota(jnp.int32, (BN, MAXZ), 1))
    oh = oh.astype(jnp.float32)
    x0 = _dot(oh, emb_ref[...])
    x0_ref[...] = x0
    f0_ref[...] = _dot(x0, w_ref[...])


def _embed(z3, emb, w0):
    return pl.pallas_call(
        _embed_body,
        grid=(NBN,),
        in_specs=[
            pl.BlockSpec((1, 1, BN), lambda i: (i, 0, 0)),
            pl.BlockSpec((MAXZ, HIDDEN), lambda i: (0, 0)),
            pl.BlockSpec((HIDDEN, HIDDEN), lambda i: (0, 0)),
        ],
        out_specs=[
            pl.BlockSpec((BN, HIDDEN), lambda i: (i, 0)),
            pl.BlockSpec((BN, HIDDEN), lambda i: (i, 0)),
        ],
        out_shape=[
            jax.ShapeDtypeStruct((N, HIDDEN), jnp.float32),
            jax.ShapeDtypeStruct((N, HIDDEN), jnp.float32),
        ],
    )(z3, emb, w0)


def _dist_body(pi_ref, pj_ref, d_ref):
    diff = (pj_ref[...] - pi_ref[...]).reshape(BE, 8)
    d2 = jnp.sum(diff * diff, axis=1) + 1e-8
    d_ref[0, 0, :] = jnp.sqrt(d2)


def _dist(pi3, pj3):
    nb = BE // PCH
    return pl.pallas_call(
        _dist_body,
        grid=(NBE,),
        in_specs=[
            pl.BlockSpec((nb, PCH, 8), lambda i: (i, 0, 0)),
            pl.BlockSpec((nb, PCH, 8), lambda i: (i, 0, 0)),
        ],
        out_specs=pl.BlockSpec((1, 1, BE), lambda i: (i, 0, 0)),
        out_shape=jax.ShapeDtypeStruct((NBE, 1, BE), jnp.float32),
    )(pi3, pj3)


def _filter_body(d_ref, w1_ref, b1_ref, w2_ref, b2_ref, wf_ref):
    d = jnp.sqrt(d_ref[0, 0, :] + 1e-8)
    step = CUTOFF / (NRBF - 1)
    coeff = -0.5 / (step * step)
    offs = lax.broadcasted_iota(jnp.int32, (BE, NRBF), 1).astype(jnp.float32) * step
    delta = d[:, None] - offs
    rbf = jnp.exp(coeff * delta * delta)
    h = _ssp(_dot(rbf, w1_ref[...]) + b1_ref[...])
    wf = _dot(h, w2_ref[...]) + b2_ref[...]
    fcut = 0.5 * (jnp.cos(jnp.pi * d / CUTOFF) + 1.0)
    fcut = fcut * (d < CUTOFF).astype(jnp.float32)
    wf_ref[...] = (wf * fcut[:, None]).reshape(BE // CHUNK, CHUNK, HIDDEN)


def _filter(d3, w1, b1, w2, b2):
    return pl.pallas_call(
        _filter_body,
        grid=(NBE,),
        in_specs=[
            pl.BlockSpec((1, 1, BE), lambda i: (i, 0, 0)),
            pl.BlockSpec((NRBF, HIDDEN), lambda i: (0, 0)),
            pl.BlockSpec((1, HIDDEN), lambda i: (0, 0)),
            pl.BlockSpec((HIDDEN, HIDDEN), lambda i: (0, 0)),
            pl.BlockSpec((1, HIDDEN), lambda i: (0, 0)),
        ],
        out_specs=pl.BlockSpec((BE // CHUNK, CHUNK, HIDDEN),
                               lambda i: (i, 0, 0)),
        out_shape=jax.ShapeDtypeStruct((E_PAD // CHUNK, CHUNK, HIDDEN),
                                       jnp.float32),
    )(d3, w1, b1, w2, b2)


def _node_body(has_next, agg_ref, x_ref, w1_ref, b1_ref, w2_ref, b2_ref,
               wn_ref, xn_ref, *maybe_f):
    h = _ssp(_dot(agg_ref[...], w1_ref[...]) + b1_ref[...])
    v = _dot(h, w2_ref[...]) + b2_ref[...]
    xn = x_ref[...] + v
    xn_ref[...] = xn
    if has_next:
        maybe_f[0][...] = _dot(xn, wn_ref[...])


def _node(agg, x, w1, b1, w2, b2, wnext, has_next):
    out_specs = [pl.BlockSpec((BN, HIDDEN), lambda i: (i, 0))]
    out_shape = [jax.ShapeDtypeStruct((N, HIDDEN), jnp.float32)]
    if has_next:
        out_specs.append(pl.BlockSpec((BN, HIDDEN), lambda i: (i, 0)))
        out_shape.append(jax.ShapeDtypeStruct((N, HIDDEN), jnp.float32))
    res = pl.pallas_call(
        functools.partial(_node_body, has_next),
        grid=(NBN,),
        in_specs=[
            pl.BlockSpec((BN, HIDDEN), lambda i: (i, 0)),
            pl.BlockSpec((BN, HIDDEN), lambda i: (i, 0)),
            pl.BlockSpec((HIDDEN, HIDDEN), lambda i: (0, 0)),
            pl.BlockSpec((1, HIDDEN), lambda i: (0, 0)),
            pl.BlockSpec((HIDDEN, HIDDEN), lambda i: (0, 0)),
            pl.BlockSpec((1, HIDDEN), lambda i: (0, 0)),
            pl.BlockSpec((HIDDEN, HIDDEN), lambda i: (0, 0)),
        ],
        out_specs=out_specs,
        out_shape=out_shape,
    )(agg, x, w1, b1, w2, b2, wnext)
    return res if has_next else (res[0], None)


def _readout_body(x_ref, b_ref, w1_ref, b1_ref, w2r_ref, b2_ref, e_ref):
    i = pl.program_id(0)
    h = _ssp(_dot(x_ref[...], w1_ref[...]) + b1_ref[...])
    yi = jnp.sum(h * w2r_ref[...], axis=1) + b2_ref[0, 0]
    bb = b_ref[0, 0, :]
    oh = (bb[:, None] == lax.broadcasted_iota(jnp.int32, (BN, 512), 1))
    contrib = _dot(yi[None, :], oh.astype(jnp.float32))

    @pl.when(i == 0)
    def _init():
        e_ref[...] = jnp.zeros_like(e_ref)

    e_ref[...] += contrib


def _readout(x, b3, aw1, aw1b, aw2r, aw2b):
    return pl.pallas_call(
        _readout_body,
        grid=(NBN,),
        in_specs=[
            pl.BlockSpec((BN, HIDDEN), lambda i: (i, 0)),
            pl.BlockSpec((1, 1, BN), lambda i: (i, 0, 0)),
            pl.BlockSpec((HIDDEN, 32), lambda i: (0, 0)),
            pl.BlockSpec((1, 32), lambda i: (0, 0)),
            pl.BlockSpec((1, 32), lambda i: (0, 0)),
            pl.BlockSpec((1, 1), lambda i: (0, 0)),
        ],
        out_specs=pl.BlockSpec((1, 512), lambda i: (0, 0)),
        out_shape=jax.ShapeDtypeStruct((1, 512), jnp.float32),
    )(x, b3, aw1, aw1b, aw2r, aw2b)


HALF = 25088
AGG_PAD = 2 * HALF
ACC_ROWS = HALF + 8
EPT = E_PAD // 16
CHUNK = 64
NCHUNK = EPT // CHUNK
GRP = 28
NGRP = NCHUNK // GRP
RPT = HALF // 16
ZROWS = 32


def _sc_gather_mul_scatter(f, wf, idxi2, idxj2):
    mesh = plsc.VectorSubcoreMesh(core_axis_name="c", subcore_axis_name="s")

    @functools.partial(
        pl.kernel, mesh=mesh,
        compiler_params=pltpu.CompilerParams(use_tc_tiling_on_sc=False,
                                             needs_layout_passes=False),
        out_type=jax.ShapeDtypeStruct((AGG_PAD, HIDDEN), jnp.float32),
        scratch_types=[
            pltpu.VMEM((GRP, CHUNK), jnp.int32),
            pltpu.VMEM((GRP, CHUNK), jnp.int32),
            pltpu.VMEM((3, CHUNK, HIDDEN), jnp.float32),
            pltpu.VMEM((3, CHUNK, HIDDEN), jnp.float32),
            pltpu.VMEM((CHUNK,), jnp.int32),
            pltpu.VMEM((ZROWS, HIDDEN), jnp.float32),
            pltpu.VMEM_SHARED((ACC_ROWS, HIDDEN), jnp.float32),
            pltpu.SemaphoreType.DMA,
            pltpu.SemaphoreType.DMA,
            pltpu.SemaphoreType.DMA,
            pltpu.SemaphoreType.DMA,
            pltpu.SemaphoreType.DMA,
            pltpu.SemaphoreType.DMA,
        ],
    )
    def k(f_hbm, wf_hbm, ii_hbm, jj_hbm, out_hbm,
          idxi_b, idxj_b, wf_b, rows_b, sidx, zbuf, acc,
          gs0, gs1, gs2, ws0, ws1, ws2):
        cid = lax.axis_index("c")
        sid = lax.axis_index("s")
        base = cid * HALF
        row0 = sid * RPT

        def zb(i, c):
            zbuf[i // 4, pl.ds((i % 4) * 16, 16)] = jnp.zeros((16,), jnp.float32)
            return c
        lax.fori_loop(0, ZROWS * 4, zb, 0)

        def zc(kk, c):
            pltpu.sync_copy(zbuf, acc.at[pl.ds(row0 + kk * ZROWS, ZROWS), :])
            return c
        lax.fori_loop(0, RPT // ZROWS, zc, 0)
        plsc.subcore_barrier()

        gsems = (gs0, gs1, gs2)
        wsems = (ws0, ws1, ws2)
        irow0 = sid * NCHUNK

        def group(g, c):
            pltpu.sync_copy(ii_hbm.at[pl.ds(irow0 + g * GRP, GRP), :], idxi_b)
            pltpu.sync_copy(jj_hbm.at[pl.ds(irow0 + g * GRP, GRP), :], idxj_b)
            c0 = irow0 + g * GRP
            handles = {}

            def issue(b):
                slot = b % 3
                hg = pltpu.async_copy(f_hbm.at[idxj_b.at[b]],
                                      rows_b.at[slot], gsems[slot])
                hw = pltpu.async_copy(wf_hbm.at[c0 + b],
                                      wf_b.at[slot], wsems[slot])
                handles[b] = (hg, hw)

            issue(0)
            issue(1)
            for b in range(GRP):
                if b + 2 < GRP:
                    issue(b + 2)
                for v in range(CHUNK // 16):
                    iv = idxi_b[b, pl.ds(v * 16, 16)]
                    loc = iv - base
                    ok = (loc >= 0) & (loc < HALF)
                    sidx[pl.ds(v * 16, 16)] = jnp.where(ok, loc, HALF)
                hg, hw = handles[b]
                hg.wait()
                hw.wait()
                slot = b % 3

                def mul(r, c2):
                    for sgm in range(4):
                        sl = pl.ds(sgm * 16, 16)
                        rows_b[slot, r, sl] = (rows_b[slot, r, sl]
                                               * wf_b[slot, r, sl])
                    return c2
                lax.fori_loop(0, CHUNK, mul, 0)
                pltpu.sync_copy(rows_b.at[slot], acc.at[sidx], add=True)
            return c
        lax.fori_loop(0, NGRP, group, 0)
        plsc.subcore_barrier()
        pltpu.sync_copy(acc.at[pl.ds(row0, RPT), :],
                        out_hbm.at[pl.ds(base + row0, RPT), :])

    return k(f, wf, idxi2, idxj2)


PCH = 128
PNCH = E_PAD // (32 * PCH)
PGRP = 14
PNGRP = PNCH // PGRP


def _take16(x, idx):
    dn = lax.GatherDimensionNumbers(offset_dims=(), collapsed_slice_dims=(0,),
                                    start_index_map=(0,))
    return lax.gather(x, idx[:, None], dn, (1,),
                      mode=lax.GatherScatterMode.PROMISE_IN_BOUNDS)


def _sc_dist2(pos8, idxi2p, idxj2p):
    mesh = plsc.VectorSubcoreMesh(core_axis_name="c", subcore_axis_name="s")

    @functools.partial(
        pl.kernel, mesh=mesh,
        compiler_params=pltpu.CompilerParams(use_tc_tiling_on_sc=False,
                                             needs_layout_passes=False),
        out_type=jax.ShapeDtypeStruct((E_PAD // PCH, PCH), jnp.float32),
        scratch_types=[
            pltpu.VMEM((PGRP, PCH), jnp.int32),
            pltpu.VMEM((PGRP, PCH), jnp.int32),
            pltpu.VMEM((2, PCH, 16), jnp.float32),
            pltpu.VMEM((2, PCH, 16), jnp.float32),
            pltpu.VMEM((PCH,), jnp.float32),
            pltpu.SemaphoreType.DMA,
            pltpu.SemaphoreType.DMA,
            pltpu.SemaphoreType.DMA,
            pltpu.SemaphoreType.DMA,
        ],
    )
    def k(pos_hbm, ii_hbm, jj_hbm, od_hbm,
          idxi_b, idxj_b, pi_b, pj_b, d2_v, s0, s1, s2, s3):
        cid = lax.axis_index("c")
        sid = lax.axis_index("s")
        wid = cid * 16 + sid
        irow0 = wid * PNCH
        isems = (s0, s1)
        jsems = (s2, s3)
        lane = lax.iota(jnp.int32, 16)
        p2 = lane ^ 2
        p1 = lane ^ 1
        pack_mask = lane == 0

        def group(g, c):
            pltpu.sync_copy(ii_hbm.at[pl.ds(irow0 + g * PGRP, PGRP), :], idxi_b)
            pltpu.sync_copy(jj_hbm.at[pl.ds(irow0 + g * PGRP, PGRP), :], idxj_b)
            c0 = irow0 + g * PGRP
            handles = {}

            def issue(b):
                slot = b % 2
                hi = pltpu.async_copy(pos_hbm.at[idxi_b.at[b]],
                                      pi_b.at[slot], isems[slot])
                hj = pltpu.async_copy(pos_hbm.at[idxj_b.at[b]],
                                      pj_b.at[slot], jsems[slot])
                handles[b] = (hi, hj)

            issue(0)
            for b in range(PGRP):
                if b + 1 < PGRP:
                    issue(b + 1)
                hi, hj = handles[b]
                hi.wait()
                hj.wait()
                slot = b % 2

                def edge2(v, c2):
                    x = pj_b[slot, v, :] - pi_b[slot, v, :]
                    s = x * x
                    s = s + _take16(s, p1)
                    s = s + _take16(s, p2)
                    tgt = lane * 0 + v
                    plsc.store_scatter(d2_v, [tgt], s, mask=pack_mask)
                    return c2
                lax.fori_loop(0, PCH, edge2, 0)
                pltpu.sync_copy(d2_v, od_hbm.at[c0 + b])
            return c
        lax.fori_loop(0, PNGRP, group, 0)

    return k(pos8, idxi2p, idxj2p)


def kernel(z, pos, batch, edge_index, emb, in2f_W, fn1_W, fn1_b, fn2_W, fn2_b,
           f2out1_W, f2out1_b, f2out2_W, f2out2_b, aw1_W, aw1_b, aw2_W, aw2_b):
    z = z.astype(jnp.int32)
    batch = batch.astype(jnp.int32)
    edge_index = edge_index.astype(jnp.int32)
    idx_i = edge_index[0]
    idx_j = edge_index[1]
    pad = E_PAD - E
    idx_i_p = jnp.pad(idx_i, (0, pad), constant_values=1 << 20)
    idx_j_p = jnp.pad(idx_j, (0, pad), constant_values=0)

    pos16 = jnp.pad(pos, ((0, 0), (0, 13)))

    idx_i_g = jnp.pad(idx_i, (0, pad), constant_values=0)
    d2 = _sc_dist2(pos16,
                   idx_i_g.reshape(E_PAD // PCH, PCH),
                   idx_j_p.reshape(E_PAD // PCH, PCH))
    d3 = d2.reshape(NBE, 1, BE)

    z3 = z.reshape(NBN, 1, BN)
    b3 = batch.reshape(NBN, 1, BN)

    x, f = _embed(z3, emb, in2f_W[0])

    idxi2 = idx_i_p.reshape(E_PAD // CHUNK, CHUNK)
    idxj2 = idx_j_p.reshape(E_PAD // CHUNK, CHUNK)

    for t in range(NINTER):
        wf = _filter(d3, fn1_W[t], fn1_b[t].reshape(1, HIDDEN),
                     fn2_W[t], fn2_b[t].reshape(1, HIDDEN))
        agg = _sc_gather_mul_scatter(f, wf, idxi2, idxj2)[:N]
        has_next = t + 1 < NINTER
        wnext = in2f_W[t + 1] if has_next else in2f_W[0]
        x, f = _node(agg, x, f2out1_W[t], f2out1_b[t].reshape(1, HIDDEN),
                     f2out2_W[t], f2out2_b[t].reshape(1, HIDDEN),
                     wnext, has_next)

    e = _readout(x, b3, aw1_W, aw1_b.reshape(1, 32),
                 aw2_W.reshape(1, 32), aw2_b.reshape(1, 1))
    return e[0, :NMOL]


# speedup vs baseline: 1.1616x; 1.0014x over previous
import functools
import math

import jax
import jax.numpy as jnp
from jax import lax
from jax.experimental import pallas as pl
from jax.experimental.pallas import tpu as pltpu
from jax.experimental.pallas import tpu_sc as plsc

N = 50000
E = 800000
NMOL = 500
HIDDEN = 64
NRBF = 50
CUTOFF = 5.0
NINTER = 3
MAXZ = 100

BN = 1000
BE = 4096
E_PAD = 802816
NBN = N // BN
NBE = E_PAD // BE

_LN2 = math.log(2.0)


def _ssp(x):
    return jnp.maximum(x, 0.0) + jnp.log(1.0 + jnp.exp(-jnp.abs(x))) - _LN2


def _dot(a, b):
    return jax.lax.dot_general(a, b, (((1,), (0,)), ((), ())),
                               preferred_element_type=jnp.float32)


def _embed_body(z_ref, emb_ref, w_ref, x0_ref, f0_ref):
    zb = z_ref[0, 0, :]
    oh = (zb[:, None] == lax.broadcasted_iota(jnp.int32, (BN, MAXZ), 1))
    oh = oh.astype(jnp.float32)
    x0 = _dot(oh, emb_ref[...])
    x0_ref[...] = x0
    f0_ref[...] = _dot(x0, w_ref[...])


def _embed(z3, emb, w0):
    return pl.pallas_call(
        _embed_body,
        grid=(NBN,),
        in_specs=[
            pl.BlockSpec((1, 1, BN), lambda i: (i, 0, 0)),
            pl.BlockSpec((MAXZ, HIDDEN), lambda i: (0, 0)),
            pl.BlockSpec((HIDDEN, HIDDEN), lambda i: (0, 0)),
        ],
        out_specs=[
            pl.BlockSpec((BN, HIDDEN), lambda i: (i, 0)),
            pl.BlockSpec((BN, HIDDEN), lambda i: (i, 0)),
        ],
        out_shape=[
            jax.ShapeDtypeStruct((N, HIDDEN), jnp.float32),
            jax.ShapeDtypeStruct((N, HIDDEN), jnp.float32),
        ],
    )(z3, emb, w0)


def _filter_body(d_ref, w1_ref, b1_ref, w2_ref, b2_ref, wf_ref):
    d = jnp.sqrt(d_ref[0, 0, :] + 1e-8)
    step = CUTOFF / (NRBF - 1)
    coeff = -0.5 / (step * step)
    offs = lax.broadcasted_iota(jnp.int32, (BE, NRBF), 1).astype(jnp.float32) * step
    delta = d[:, None] - offs
    rbf = jnp.exp(coeff * delta * delta)
    h = _ssp(_dot(rbf, w1_ref[...]) + b1_ref[...])
    wf = _dot(h, w2_ref[...]) + b2_ref[...]
    fcut = 0.5 * (jnp.cos(jnp.pi * d / CUTOFF) + 1.0)
    fcut = fcut * (d < CUTOFF).astype(jnp.float32)
    wf_ref[...] = (wf * fcut[:, None]).reshape(BE // CHUNK, CHUNK, HIDDEN)


def _filter(d3, w1, b1, w2, b2):
    return pl.pallas_call(
        _filter_body,
        grid=(NBE,),
        in_specs=[
            pl.BlockSpec((1, 1, BE), lambda i: (i, 0, 0)),
            pl.BlockSpec((NRBF, HIDDEN), lambda i: (0, 0)),
            pl.BlockSpec((1, HIDDEN), lambda i: (0, 0)),
            pl.BlockSpec((HIDDEN, HIDDEN), lambda i: (0, 0)),
            pl.BlockSpec((1, HIDDEN), lambda i: (0, 0)),
        ],
        out_specs=pl.BlockSpec((BE // CHUNK, CHUNK, HIDDEN),
                               lambda i: (i, 0, 0)),
        out_shape=jax.ShapeDtypeStruct((E_PAD // CHUNK, CHUNK, HIDDEN),
                                       jnp.float32),
    )(d3, w1, b1, w2, b2)


def _node_body(has_next, agg_ref, x_ref, w1_ref, b1_ref, w2_ref, b2_ref,
               wn_ref, xn_ref, *maybe_f):
    h = _ssp(_dot(agg_ref[...], w1_ref[...]) + b1_ref[...])
    v = _dot(h, w2_ref[...]) + b2_ref[...]
    xn = x_ref[...] + v
    xn_ref[...] = xn
    if has_next:
        maybe_f[0][...] = _dot(xn, wn_ref[...])


def _node(agg, x, w1, b1, w2, b2, wnext, has_next):
    out_specs = [pl.BlockSpec((BN, HIDDEN), lambda i: (i, 0))]
    out_shape = [jax.ShapeDtypeStruct((N, HIDDEN), jnp.float32)]
    if has_next:
        out_specs.append(pl.BlockSpec((BN, HIDDEN), lambda i: (i, 0)))
        out_shape.append(jax.ShapeDtypeStruct((N, HIDDEN), jnp.float32))
    res = pl.pallas_call(
        functools.partial(_node_body, has_next),
        grid=(NBN,),
        in_specs=[
            pl.BlockSpec((BN, HIDDEN), lambda i: (i, 0)),
            pl.BlockSpec((BN, HIDDEN), lambda i: (i, 0)),
            pl.BlockSpec((HIDDEN, HIDDEN), lambda i: (0, 0)),
            pl.BlockSpec((1, HIDDEN), lambda i: (0, 0)),
            pl.BlockSpec((HIDDEN, HIDDEN), lambda i: (0, 0)),
            pl.BlockSpec((1, HIDDEN), lambda i: (0, 0)),
            pl.BlockSpec((HIDDEN, HIDDEN), lambda i: (0, 0)),
        ],
        out_specs=out_specs,
        out_shape=out_shape,
    )(agg, x, w1, b1, w2, b2, wnext)
    return res if has_next else (res[0], None)


def _readout_body(x_ref, b_ref, w1_ref, b1_ref, w2r_ref, b2_ref, e_ref):
    i = pl.program_id(0)
    h = _ssp(_dot(x_ref[...], w1_ref[...]) + b1_ref[...])
    yi = jnp.sum(h * w2r_ref[...], axis=1) + b2_ref[0, 0]
    bb = b_ref[0, 0, :]
    oh = (bb[:, None] == lax.broadcasted_iota(jnp.int32, (BN, 512), 1))
    contrib = _dot(yi[None, :], oh.astype(jnp.float32))

    @pl.when(i == 0)
    def _init():
        e_ref[...] = jnp.zeros_like(e_ref)

    e_ref[...] += contrib


def _readout(x, b3, aw1, aw1b, aw2r, aw2b):
    return pl.pallas_call(
        _readout_body,
        grid=(NBN,),
        in_specs=[
            pl.BlockSpec((BN, HIDDEN), lambda i: (i, 0)),
            pl.BlockSpec((1, 1, BN), lambda i: (i, 0, 0)),
            pl.BlockSpec((HIDDEN, 32), lambda i: (0, 0)),
            pl.BlockSpec((1, 32), lambda i: (0, 0)),
            pl.BlockSpec((1, 32), lambda i: (0, 0)),
            pl.BlockSpec((1, 1), lambda i: (0, 0)),
        ],
        out_specs=pl.BlockSpec((1, 512), lambda i: (0, 0)),
        out_shape=jax.ShapeDtypeStruct((1, 512), jnp.float32),
    )(x, b3, aw1, aw1b, aw2r, aw2b)


HALF = 25088
AGG_PAD = 2 * HALF
ACC_ROWS = HALF + 8
EPT = E_PAD // 16
CHUNK = 64
NCHUNK = EPT // CHUNK
GRP = 28
NGRP = NCHUNK // GRP
RPT = HALF // 16
ZROWS = 32


def _sc_gather_mul_scatter(f, wf, idxi2, idxj2):
    mesh = plsc.VectorSubcoreMesh(core_axis_name="c", subcore_axis_name="s")

    @functools.partial(
        pl.kernel, mesh=mesh,
        compiler_params=pltpu.CompilerParams(use_tc_tiling_on_sc=False,
                                             needs_layout_passes=False),
        out_type=jax.ShapeDtypeStruct((AGG_PAD, HIDDEN), jnp.float32),
        scratch_types=[
            pltpu.VMEM((GRP, CHUNK), jnp.int32),
            pltpu.VMEM((GRP, CHUNK), jnp.int32),
            pltpu.VMEM((3, CHUNK, HIDDEN), jnp.float32),
            pltpu.VMEM((3, CHUNK, HIDDEN), jnp.float32),
            pltpu.VMEM((CHUNK,), jnp.int32),
            pltpu.VMEM((ZROWS, HIDDEN), jnp.float32),
            pltpu.VMEM_SHARED((ACC_ROWS, HIDDEN), jnp.float32),
            pltpu.SemaphoreType.DMA,
            pltpu.SemaphoreType.DMA,
            pltpu.SemaphoreType.DMA,
            pltpu.SemaphoreType.DMA,
            pltpu.SemaphoreType.DMA,
            pltpu.SemaphoreType.DMA,
        ],
    )
    def k(f_hbm, wf_hbm, ii_hbm, jj_hbm, out_hbm,
          idxi_b, idxj_b, wf_b, rows_b, sidx, zbuf, acc,
          gs0, gs1, gs2, ws0, ws1, ws2):
        cid = lax.axis_index("c")
        sid = lax.axis_index("s")
        base = cid * HALF
        row0 = sid * RPT

        def zb(i, c):
            zbuf[i // 4, pl.ds((i % 4) * 16, 16)] = jnp.zeros((16,), jnp.float32)
            return c
        lax.fori_loop(0, ZROWS * 4, zb, 0)

        def zc(kk, c):
            pltpu.sync_copy(zbuf, acc.at[pl.ds(row0 + kk * ZROWS, ZROWS), :])
            return c
        lax.fori_loop(0, RPT // ZROWS, zc, 0)
        plsc.subcore_barrier()

        gsems = (gs0, gs1, gs2)
        wsems = (ws0, ws1, ws2)
        irow0 = sid * NCHUNK

        def group(g, c):
            pltpu.sync_copy(ii_hbm.at[pl.ds(irow0 + g * GRP, GRP), :], idxi_b)
            pltpu.sync_copy(jj_hbm.at[pl.ds(irow0 + g * GRP, GRP), :], idxj_b)
            c0 = irow0 + g * GRP
            handles = {}

            def issue(b):
                slot = b % 3
                hg = pltpu.async_copy(f_hbm.at[idxj_b.at[b]],
                                      rows_b.at[slot], gsems[slot])
                hw = pltpu.async_copy(wf_hbm.at[c0 + b],
                                      wf_b.at[slot], wsems[slot])
                handles[b] = (hg, hw)

            issue(0)
            issue(1)
            for b in range(GRP):
                if b + 2 < GRP:
                    issue(b + 2)
                for v in range(CHUNK // 16):
                    iv = idxi_b[b, pl.ds(v * 16, 16)]
                    loc = iv - base
                    ok = (loc >= 0) & (loc < HALF)
                    sidx[pl.ds(v * 16, 16)] = jnp.where(ok, loc, HALF)
                hg, hw = handles[b]
                hg.wait()
                hw.wait()
                slot = b % 3

                def mul(r, c2):
                    for sgm in range(4):
                        sl = pl.ds(sgm * 16, 16)
                        rows_b[slot, r, sl] = (rows_b[slot, r, sl]
                                               * wf_b[slot, r, sl])
                    return c2
                lax.fori_loop(0, CHUNK, mul, 0)
                pltpu.sync_copy(rows_b.at[slot], acc.at[sidx], add=True)
            return c
        lax.fori_loop(0, NGRP, group, 0)
        plsc.subcore_barrier()
        pltpu.sync_copy(acc.at[pl.ds(row0, RPT), :],
                        out_hbm.at[pl.ds(base + row0, RPT), :])

    return k(f, wf, idxi2, idxj2)


PCH = 128
PNCH = E_PAD // (32 * PCH)
PGRP = 14
PNGRP = PNCH // PGRP


def _take16(x, idx):
    dn = lax.GatherDimensionNumbers(offset_dims=(), collapsed_slice_dims=(0,),
                                    start_index_map=(0,))
    return lax.gather(x, idx[:, None], dn, (1,),
                      mode=lax.GatherScatterMode.PROMISE_IN_BOUNDS)


def _sc_dist2(pos8, idxi2p, idxj2p):
    mesh = plsc.VectorSubcoreMesh(core_axis_name="c", subcore_axis_name="s")

    @functools.partial(
        pl.kernel, mesh=mesh,
        compiler_params=pltpu.CompilerParams(use_tc_tiling_on_sc=False,
                                             needs_layout_passes=False),
        out_type=jax.ShapeDtypeStruct((E_PAD // PCH, PCH), jnp.float32),
        scratch_types=[
            pltpu.VMEM((PGRP, PCH), jnp.int32),
            pltpu.VMEM((PGRP, PCH), jnp.int32),
            pltpu.VMEM((2, PCH, 16), jnp.float32),
            pltpu.VMEM((2, PCH, 16), jnp.float32),
            pltpu.VMEM((PCH,), jnp.float32),
            pltpu.SemaphoreType.DMA,
            pltpu.SemaphoreType.DMA,
            pltpu.SemaphoreType.DMA,
            pltpu.SemaphoreType.DMA,
        ],
    )
    def k(pos_hbm, ii_hbm, jj_hbm, od_hbm,
          idxi_b, idxj_b, pi_b, pj_b, d2_v, s0, s1, s2, s3):
        cid = lax.axis_index("c")
        sid = lax.axis_index("s")
        wid = cid * 16 + sid
        irow0 = wid * PNCH
        isems = (s0, s1)
        jsems = (s2, s3)
        lane = lax.iota(jnp.int32, 16)
        p2 = lane ^ 2
        p1 = lane ^ 1
        pack_mask = lane == 0

        def group(g, c):
            pltpu.sync_copy(ii_hbm.at[pl.ds(irow0 + g * PGRP, PGRP), :], idxi_b)
            pltpu.sync_copy(jj_hbm.at[pl.ds(irow0 + g * PGRP, PGRP), :], idxj_b)
            c0 = irow0 + g * PGRP
            handles = {}

            def issue(b):
                slot = b % 2
                hi = pltpu.async_copy(pos_hbm.at[idxi_b.at[b]],
                                      pi_b.at[slot], isems[slot])
                hj = pltpu.async_copy(pos_hbm.at[idxj_b.at[b]],
                                      pj_b.at[slot], jsems[slot])
                handles[b] = (hi, hj)

            issue(0)
            for b in range(PGRP):
                if b + 1 < PGRP:
                    issue(b + 1)
                hi, hj = handles[b]
                hi.wait()
                hj.wait()
                slot = b % 2

                def edge2(v, c2):
                    x = pj_b[slot, v, :] - pi_b[slot, v, :]
                    s = x * x
                    s = s + _take16(s, p1)
                    s = s + _take16(s, p2)
                    tgt = lane * 0 + v
                    plsc.store_scatter(d2_v, [tgt], s, mask=pack_mask)
                    return c2
                lax.fori_loop(0, PCH, edge2, 0)
                pltpu.sync_copy(d2_v, od_hbm.at[c0 + b])
            return c
        lax.fori_loop(0, PNGRP, group, 0)

    return k(pos8, idxi2p, idxj2p)


def kernel(z, pos, batch, edge_index, emb, in2f_W, fn1_W, fn1_b, fn2_W, fn2_b,
           f2out1_W, f2out1_b, f2out2_W, f2out2_b, aw1_W, aw1_b, aw2_W, aw2_b):
    z = z.astype(jnp.int32)
    batch = batch.astype(jnp.int32)
    edge_index = edge_index.astype(jnp.int32)
    idx_i = edge_index[0]
    idx_j = edge_index[1]
    pad = E_PAD - E
    idx_i_p = jnp.pad(idx_i, (0, pad), constant_values=1 << 20)
    idx_j_p = jnp.pad(idx_j, (0, pad), constant_values=0)

    pos16 = jnp.pad(pos, ((0, 0), (0, 13)))

    idx_i_g = jnp.pad(idx_i, (0, pad), constant_values=0)
    d2 = _sc_dist2(pos16,
                   idx_i_g.reshape(E_PAD // PCH, PCH),
                   idx_j_p.reshape(E_PAD // PCH, PCH))
    d3 = d2.reshape(NBE, 1, BE)

    z3 = z.reshape(NBN, 1, BN)
    b3 = batch.reshape(NBN, 1, BN)

    x, f = _embed(z3, emb, in2f_W[0])

    idxi2 = idx_i_p.reshape(E_PAD // CHUNK, CHUNK)
    idxj2 = idx_j_p.reshape(E_PAD // CHUNK, CHUNK)

    for t in range(NINTER):
        wf = _filter(d3, fn1_W[t], fn1_b[t].reshape(1, HIDDEN),
                     fn2_W[t], fn2_b[t].reshape(1, HIDDEN))
        agg = _sc_gather_mul_scatter(f, wf, idxi2, idxj2)[:N]
        has_next = t + 1 < NINTER
        wnext = in2f_W[t + 1] if has_next else in2f_W[0]
        x, f = _node(agg, x, f2out1_W[t], f2out1_b[t].reshape(1, HIDDEN),
                     f2out2_W[t], f2out2_b[t].reshape(1, HIDDEN),
                     wnext, has_next)

    e = _readout(x, b3, aw1_W, aw1_b.reshape(1, 32),
                 aw2_W.reshape(1, 32), aw2_b.reshape(1, 1))
    return e[0, :NMOL]
